# baseline (speedup 1.0000x reference)
import functools

import jax
import jax.numpy as jnp
from jax import lax
from jax.experimental import pallas as pl
from jax.experimental.pallas import tpu as pltpu
from jax.experimental.pallas import tpu_sc as plsc

N_NODES = 100000
N_EDGES = 6400000
D = 3
BATCH = 128
ROWS = N_EDGES // BATCH
NC, NS = 2, 16
NW = NC * NS
K = 8
GROUPS = ROWS // K
GPW = GROUPS // NW
GREM = GROUPS - GPW * NW
N_PAD = 100096
BN = N_PAD // NS

_mesh = plsc.VectorSubcoreMesh(core_axis_name="c", subcore_axis_name="s")


def _worker_blocks(c, s):
    w = s * NC + c
    start = (w * GPW + jnp.minimum(w, GREM)) * K
    nblocks = GPW + jnp.where(w < GREM, 1, 0)
    return start, nblocks


@functools.partial(
    pl.kernel,
    out_type=jax.ShapeDtypeStruct((NC * N_PAD,), jnp.float32),
    mesh=_mesh,
    scratch_types=[
        pltpu.VMEM((3, K, BATCH), jnp.int32),
        pltpu.VMEM((BATCH,), jnp.float32),
        pltpu.VMEM((BN,), jnp.float32),
        pltpu.VMEM((K, BATCH), jnp.float32),
        pltpu.VMEM_SHARED((N_PAD,), jnp.float32),
        pltpu.SemaphoreType.DMA,
        pltpu.SemaphoreType.DMA,
    ],
)
def _degree_pass(dst2d, zeros2, zeros_n, out, dst_v, ones_v, stage, dbuf,
                 acc, isem, ssem):
    c = lax.axis_index("c")
    s = lax.axis_index("s")
    for j in range(BATCH // 16):
        ones_v[pl.ds(j * 16, 16)] = jnp.ones((16,), jnp.float32)
    pltpu.sync_copy(zeros_n.at[pl.ds(s * BN, BN)], stage)
    pltpu.sync_copy(stage, acc.at[pl.ds(s * BN, BN)])
    plsc.subcore_barrier()

    start, nblocks = _worker_blocks(c, s)

    def drain_scatters():
        pltpu.make_async_copy(zeros2, dbuf, ssem).wait()

    pltpu.sync_copy(dst2d.at[pl.ds(start, K)], dst_v.at[0])

    def block(i, carry):
        q = lax.rem(i, 3)

        @pl.when(i >= 2)
        def _():
            drain_scatters()

        @pl.when(i >= 1)
        def _():
            pltpu.make_async_copy(
                dst2d.at[pl.ds(start, K)], dst_v.at[q], isem).wait()

        @pl.when(i + 1 < nblocks)
        def _():
            q1 = lax.rem(i + 1, 3)
            pltpu.async_copy(dst2d.at[pl.ds(start + (i + 1) * K, K)],
                             dst_v.at[q1], isem)

        for j in range(K):
            pltpu.async_copy(ones_v, acc.at[dst_v.at[q, j]], ssem, add=True)
        return carry

    lax.fori_loop(0, nblocks, block, 0)
    drain_scatters()
    drain_scatters()
    plsc.subcore_barrier()
    pltpu.sync_copy(acc.at[pl.ds(s * BN, BN)], stage)
    pltpu.sync_copy(stage, out.at[pl.ds(c * N_PAD + s * BN, BN)])


@functools.partial(
    pl.kernel,
    out_type=[jax.ShapeDtypeStruct((NC * N_PAD,), jnp.float32)
              for _ in range(D)],
    mesh=_mesh,
    scratch_types=[
        pltpu.VMEM((3, K, BATCH), jnp.int32),
        pltpu.VMEM((3, K, BATCH), jnp.int32),
        pltpu.VMEM((2, D, K, BATCH), jnp.float32),
        pltpu.VMEM((BN,), jnp.float32),
        pltpu.VMEM_SHARED((N_PAD,), jnp.float32),
        pltpu.VMEM_SHARED((N_PAD,), jnp.float32),
        pltpu.VMEM_SHARED((N_PAD,), jnp.float32),
        pltpu.SemaphoreType.DMA,
        pltpu.SemaphoreType.DMA,
        pltpu.SemaphoreType.DMA,
    ],
)
def _scatter_pass(src2d, dst2d, t0, t1, t2, zeros3, zeros_n, o0, o1, o2,
                  src_v, dst_v, vals_v, stage, a0, a1, a2, isem, gsem, ssem):
    c = lax.axis_index("c")
    s = lax.axis_index("s")
    tables = (t0, t1, t2)
    accs = (a0, a1, a2)
    outs = (o0, o1, o2)
    for acc in accs:
        pltpu.sync_copy(zeros_n.at[pl.ds(s * BN, BN)], stage)
        pltpu.sync_copy(stage, acc.at[pl.ds(s * BN, BN)])
    plsc.subcore_barrier()

    start, nblocks = _worker_blocks(c, s)

    def drain_scatters(p):
        pltpu.make_async_copy(zeros3, vals_v.at[p], ssem).wait()

    pltpu.sync_copy(src2d.at[pl.ds(start, K)], src_v.at[0])
    pltpu.sync_copy(dst2d.at[pl.ds(start, K)], dst_v.at[0])

    def block(i, carry):
        p = lax.rem(i, 2)
        q = lax.rem(i, 3)

        @pl.when(i >= 2)
        def _():
            drain_scatters(p)

        @pl.when(i >= 1)
        def _():
            pltpu.make_async_copy(
                src2d.at[pl.ds(start, K)], src_v.at[q], isem).wait()
            pltpu.make_async_copy(
                dst2d.at[pl.ds(start, K)], dst_v.at[q], isem).wait()

        @pl.when(i + 1 < nblocks)
        def _():
            q1 = lax.rem(i + 1, 3)
            r1 = start + (i + 1) * K
            pltpu.async_copy(src2d.at[pl.ds(r1, K)], src_v.at[q1], isem)
            pltpu.async_copy(dst2d.at[pl.ds(r1, K)], dst_v.at[q1], isem)

        for j in range(K):
            for d in range(D):
                pltpu.async_copy(tables[d].at[src_v.at[q, j]],
                                 vals_v.at[p, d, j], gsem)
        pltpu.make_async_copy(zeros3, vals_v.at[p], gsem).wait()
        for j in range(K):
            for d in range(D):
                pltpu.async_copy(vals_v.at[p, d, j],
                                 accs[d].at[dst_v.at[q, j]], ssem, add=True)
        return carry

    lax.fori_loop(0, nblocks, block, 0)
    drain_scatters(lax.rem(nblocks, 2))
    drain_scatters(lax.rem(nblocks + 1, 2))
    plsc.subcore_barrier()
    for d in range(D):
        pltpu.sync_copy(accs[d].at[pl.ds(s * BN, BN)], stage)
        pltpu.sync_copy(stage, outs[d].at[pl.ds(c * N_PAD + s * BN, BN)])


def _sweep(src2d, dst2d, table, zeros3, zeros_n):
    cols = [jnp.asarray(table[:, d]) for d in range(D)]
    parts = _scatter_pass(src2d, dst2d, *cols, zeros3, zeros_n)
    return jnp.stack(
        [p.reshape(NC, N_PAD)[0, :N_NODES] + p.reshape(NC, N_PAD)[1, :N_NODES]
         for p in parts], axis=1)


def kernel(subgraph, feat, send_map, recv_map, rank, size, W1, b1, W2, b2):
    src2d = subgraph[0].reshape(ROWS, BATCH)
    dst2d = subgraph[1].reshape(ROWS, BATCH)
    zeros_n = jnp.zeros((N_PAD,), jnp.float32)
    zeros2 = jnp.zeros((K, BATCH), jnp.float32)
    zeros3 = jnp.zeros((D, K, BATCH), jnp.float32)

    degp = _degree_pass(dst2d, zeros2, zeros_n).reshape(NC, N_PAD)
    deg = jnp.maximum(degp[0, :N_NODES] + degp[1, :N_NODES], 1.0)
    rdeg = lax.rsqrt(deg)[:, None]

    t1 = (feat * rdeg) @ W1
    agg1 = _sweep(src2d, dst2d, t1, zeros3, zeros_n)
    h = jnp.maximum(agg1 * rdeg + b1, 0.0)

    t2 = (h * rdeg) @ W2
    agg2 = _sweep(src2d, dst2d, t2, zeros3, zeros_n)
    out = agg2 * rdeg + b2
    return out


# speedup vs baseline: 1.7003x; 1.7003x over previous
import functools

import jax
import jax.numpy as jnp
from jax import lax
from jax.experimental import pallas as pl
from jax.experimental.pallas import tpu as pltpu
from jax.experimental.pallas import tpu_sc as plsc

N_NODES = 100000
N_EDGES = 6400000
D = 3
BATCH = 128
ROWS = N_EDGES // BATCH
NC, NS = 2, 16
NW = NC * NS
K = 8
GROUPS = ROWS // K
GPW = GROUPS // NW
GREM = GROUPS - GPW * NW
N_PAD = 100096
BN = N_PAD // NS

_mesh = plsc.VectorSubcoreMesh(core_axis_name="c", subcore_axis_name="s")


def _worker_blocks(c, s):
    w = s * NC + c
    start = (w * GPW + jnp.minimum(w, GREM)) * K
    nblocks = GPW + jnp.where(w < GREM, 1, 0)
    return start, nblocks


@functools.partial(
    pl.kernel,
    out_type=jax.ShapeDtypeStruct((NC * N_PAD,), jnp.float32),
    mesh=_mesh,
    scratch_types=[
        pltpu.VMEM((3, K, BATCH), jnp.int32),
        pltpu.VMEM((BATCH,), jnp.float32),
        pltpu.VMEM((BN,), jnp.float32),
        pltpu.VMEM((K, BATCH), jnp.float32),
        pltpu.VMEM_SHARED((N_PAD,), jnp.float32),
        pltpu.SemaphoreType.DMA,
        pltpu.SemaphoreType.DMA,
    ],
)
def _degree_pass(dst2d, zeros2, zeros_n, out, dst_v, ones_v, stage, dbuf,
                 acc, isem, ssem):
    c = lax.axis_index("c")
    s = lax.axis_index("s")
    for j in range(BATCH // 16):
        ones_v[pl.ds(j * 16, 16)] = jnp.ones((16,), jnp.float32)
    pltpu.sync_copy(zeros_n.at[pl.ds(s * BN, BN)], stage)
    pltpu.sync_copy(stage, acc.at[pl.ds(s * BN, BN)])
    plsc.subcore_barrier()

    start, nblocks = _worker_blocks(c, s)

    def drain_scatters():
        pltpu.make_async_copy(zeros2, dbuf, ssem).wait()

    pltpu.sync_copy(dst2d.at[pl.ds(start, K)], dst_v.at[0])

    def block(i, carry):
        q = lax.rem(i, 3)

        @pl.when(i >= 2)
        def _():
            drain_scatters()

        @pl.when(i >= 1)
        def _():
            pltpu.make_async_copy(
                dst2d.at[pl.ds(start, K)], dst_v.at[q], isem).wait()

        @pl.when(i + 1 < nblocks)
        def _():
            q1 = lax.rem(i + 1, 3)
            pltpu.async_copy(dst2d.at[pl.ds(start + (i + 1) * K, K)],
                             dst_v.at[q1], isem)

        for j in range(K):
            pltpu.async_copy(ones_v, acc.at[dst_v.at[q, j]], ssem, add=True)
        return carry

    lax.fori_loop(0, nblocks, block, 0)
    drain_scatters()
    drain_scatters()
    plsc.subcore_barrier()
    pltpu.sync_copy(acc.at[pl.ds(s * BN, BN)], stage)
    pltpu.sync_copy(stage, out.at[pl.ds(c * N_PAD + s * BN, BN)])


@functools.partial(
    pl.kernel,
    out_type=[jax.ShapeDtypeStruct((NC * N_PAD,), jnp.float32)
              for _ in range(D)],
    mesh=_mesh,
    scratch_types=[
        pltpu.VMEM((3, K, BATCH), jnp.int32),
        pltpu.VMEM((3, K, BATCH), jnp.int32),
        pltpu.VMEM((2, D, K, BATCH), jnp.float32),
        pltpu.VMEM((BN,), jnp.float32),
        pltpu.VMEM_SHARED((N_PAD,), jnp.float32),
        pltpu.VMEM_SHARED((N_PAD,), jnp.float32),
        pltpu.VMEM_SHARED((N_PAD,), jnp.float32),
        pltpu.VMEM_SHARED((N_PAD,), jnp.float32),
        pltpu.VMEM_SHARED((N_PAD,), jnp.float32),
        pltpu.VMEM_SHARED((N_PAD,), jnp.float32),
        pltpu.SemaphoreType.DMA,
        pltpu.SemaphoreType.DMA,
        pltpu.SemaphoreType.DMA,
    ],
)
def _scatter_pass(src2d, dst2d, t0, t1, t2, zeros3, zeros_n, o0, o1, o2,
                  src_v, dst_v, vals_v, stage, a0, a1, a2, b0, b1, b2,
                  isem, gsem, ssem):
    c = lax.axis_index("c")
    s = lax.axis_index("s")
    accs = (a0, a1, a2)
    tables = (b0, b1, b2)
    outs = (o0, o1, o2)
    for acc in accs:
        pltpu.sync_copy(zeros_n.at[pl.ds(s * BN, BN)], stage)
        pltpu.sync_copy(stage, acc.at[pl.ds(s * BN, BN)])
    for src_t, dst_t in zip((t0, t1, t2), tables):
        pltpu.sync_copy(src_t.at[pl.ds(s * BN, BN)], stage)
        pltpu.sync_copy(stage, dst_t.at[pl.ds(s * BN, BN)])
    plsc.subcore_barrier()

    start, nblocks = _worker_blocks(c, s)

    def drain_scatters(p):
        pltpu.make_async_copy(zeros3, vals_v.at[p], ssem).wait()

    pltpu.sync_copy(src2d.at[pl.ds(start, K)], src_v.at[0])
    pltpu.sync_copy(dst2d.at[pl.ds(start, K)], dst_v.at[0])

    def block(i, carry):
        p = lax.rem(i, 2)
        q = lax.rem(i, 3)

        @pl.when(i >= 2)
        def _():
            drain_scatters(p)

        @pl.when(i >= 1)
        def _():
            pltpu.make_async_copy(
                src2d.at[pl.ds(start, K)], src_v.at[q], isem).wait()
            pltpu.make_async_copy(
                dst2d.at[pl.ds(start, K)], dst_v.at[q], isem).wait()

        @pl.when(i + 1 < nblocks)
        def _():
            q1 = lax.rem(i + 1, 3)
            r1 = start + (i + 1) * K
            pltpu.async_copy(src2d.at[pl.ds(r1, K)], src_v.at[q1], isem)
            pltpu.async_copy(dst2d.at[pl.ds(r1, K)], dst_v.at[q1], isem)

        for j in range(K):
            for d in range(D):
                pltpu.async_copy(tables[d].at[src_v.at[q, j]],
                                 vals_v.at[p, d, j], gsem)
        pltpu.make_async_copy(zeros3, vals_v.at[p], gsem).wait()
        for j in range(K):
            for d in range(D):
                pltpu.async_copy(vals_v.at[p, d, j],
                                 accs[d].at[dst_v.at[q, j]], ssem, add=True)
        return carry

    lax.fori_loop(0, nblocks, block, 0)
    drain_scatters(lax.rem(nblocks, 2))
    drain_scatters(lax.rem(nblocks + 1, 2))
    plsc.subcore_barrier()
    for d in range(D):
        pltpu.sync_copy(accs[d].at[pl.ds(s * BN, BN)], stage)
        pltpu.sync_copy(stage, outs[d].at[pl.ds(c * N_PAD + s * BN, BN)])


def _sweep(src2d, dst2d, table, zeros3, zeros_n):
    tp = jnp.pad(table, ((0, N_PAD - N_NODES), (0, 0)))
    cols = [jnp.asarray(tp[:, d]) for d in range(D)]
    parts = _scatter_pass(src2d, dst2d, *cols, zeros3, zeros_n)
    return jnp.stack(
        [p.reshape(NC, N_PAD)[0, :N_NODES] + p.reshape(NC, N_PAD)[1, :N_NODES]
         for p in parts], axis=1)


def kernel(subgraph, feat, send_map, recv_map, rank, size, W1, b1, W2, b2):
    src2d = subgraph[0].reshape(ROWS, BATCH)
    dst2d = subgraph[1].reshape(ROWS, BATCH)
    zeros_n = jnp.zeros((N_PAD,), jnp.float32)
    zeros2 = jnp.zeros((K, BATCH), jnp.float32)
    zeros3 = jnp.zeros((D, K, BATCH), jnp.float32)

    degp = _degree_pass(dst2d, zeros2, zeros_n).reshape(NC, N_PAD)
    deg = jnp.maximum(degp[0, :N_NODES] + degp[1, :N_NODES], 1.0)
    rdeg = lax.rsqrt(deg)[:, None]

    t1 = (feat * rdeg) @ W1
    agg1 = _sweep(src2d, dst2d, t1, zeros3, zeros_n)
    h = jnp.maximum(agg1 * rdeg + b1, 0.0)

    t2 = (h * rdeg) @ W2
    agg2 = _sweep(src2d, dst2d, t2, zeros3, zeros_n)
    out = agg2 * rdeg + b2
    return out
